# baseline (speedup 1.0000x reference)
import functools

import jax
import jax.numpy as jnp
from jax import lax
from jax.experimental import pallas as pl
from jax.experimental.pallas import tpu as pltpu


def _gather_body(idx_ref, t_ref, o_ref, sem):
    blk = o_ref.shape[0]

    def issue(i, c):
        r = idx_ref[0, 0, i]
        pltpu.make_async_copy(t_ref.at[r], o_ref.at[i], sem).start()
        return c

    lax.fori_loop(0, blk, issue, 0)

    def drain(i, c):
        pltpu.make_async_copy(t_ref.at[0], o_ref.at[i], sem).wait()
        return c

    lax.fori_loop(0, blk, drain, 0)


def _tc_gather(table, idx, blk=2048):
    B = idx.shape[0]
    D = table.shape[1]
    nblk = B // blk
    idx3 = idx.reshape(nblk, 1, blk)
    return pl.pallas_call(
        _gather_body,
        grid=(nblk,),
        in_specs=[
            pl.BlockSpec((1, 1, blk), lambda i: (i, 0, 0),
                         memory_space=pltpu.SMEM),
            pl.BlockSpec(memory_space=pl.ANY),
        ],
        out_specs=pl.BlockSpec((blk, D), lambda i: (i, 0)),
        out_shape=jax.ShapeDtypeStruct((B, D), jnp.float32),
        scratch_shapes=[pltpu.SemaphoreType.DMA],
    )(idx3, table)


def _mlp_body(m_ref, gidf_ref, gt_ref, w1_ref, b1_ref, w2_ref, b2_ref,
              w3_ref, b3_ref, o_ref):
    m_emb = m_ref[...]
    blk = m_emb.shape[0]
    P = gt_ref.shape[0]
    lane = lax.broadcasted_iota(jnp.int32, (blk, P), 1).astype(jnp.float32)
    onehot = jnp.where(lane == gidf_ref[...], 1.0, 0.0)
    g_emb = jnp.dot(onehot, gt_ref[...], preferred_element_type=jnp.float32)
    x = m_emb + g_emb
    h = jnp.dot(x, w1_ref[...], preferred_element_type=jnp.float32)
    h = jnp.maximum(h + b1_ref[...], 0.0)
    h = jnp.dot(h, w2_ref[...], preferred_element_type=jnp.float32)
    h = jnp.maximum(h + b2_ref[...], 0.0)
    logits = jnp.dot(h, w3_ref[...], preferred_element_type=jnp.float32)
    logits = logits + b3_ref[...]
    mx = jnp.max(logits, axis=-1, keepdims=True)
    e = jnp.exp(logits - mx)
    o_ref[...] = e / jnp.sum(e, axis=-1, keepdims=True)


def _mlp_softmax(m_emb, gid_f, gt_pad, W1, b1, W2, b2, W3p, b3p, blk=2048):
    B, D = m_emb.shape
    GP = gt_pad.shape[0]
    H1 = W1.shape[1]
    H2 = W2.shape[1]
    P = W3p.shape[1]
    return pl.pallas_call(
        _mlp_body,
        grid=(B // blk,),
        in_specs=[
            pl.BlockSpec((blk, D), lambda i: (i, 0)),
            pl.BlockSpec((blk, 1), lambda i: (i, 0)),
            pl.BlockSpec((GP, D), lambda i: (0, 0)),
            pl.BlockSpec((D, H1), lambda i: (0, 0)),
            pl.BlockSpec((1, H1), lambda i: (0, 0)),
            pl.BlockSpec((H1, H2), lambda i: (0, 0)),
            pl.BlockSpec((1, H2), lambda i: (0, 0)),
            pl.BlockSpec((H2, P), lambda i: (0, 0)),
            pl.BlockSpec((1, P), lambda i: (0, 0)),
        ],
        out_specs=pl.BlockSpec((blk, P), lambda i: (i, 0)),
        out_shape=jax.ShapeDtypeStruct((B, P), jnp.float32),
    )(m_emb, gid_f, gt_pad, W1, b1.reshape(1, H1), W2, b2.reshape(1, H2),
      W3p, b3p.reshape(1, P))


def kernel(movie_id, genre_id, movie_table, genre_table, W1, b1, W2, b2, W3,
           b3):
    B = movie_id.shape[0]
    m_emb = _tc_gather(movie_table, movie_id.astype(jnp.int32))
    gid_f = genre_id.astype(jnp.float32).reshape(B, 1)
    G = genre_table.shape[0]
    gt_pad = jnp.pad(genre_table, ((0, (-G) % 128), (0, 0)))
    NG = W3.shape[1]
    pad = (-NG) % 128
    W3p = jnp.pad(W3, ((0, 0), (0, pad)))
    b3p = jnp.concatenate([b3, jnp.full((pad,), -1e30, dtype=b3.dtype)])
    out = _mlp_softmax(m_emb, gid_f, gt_pad, W1, b1, W2, b2, W3p, b3p)
    return out[:, :NG]


# speedup vs baseline: 2.2163x; 2.2163x over previous
import dataclasses
import functools

import jax
import jax.numpy as jnp
from jax import lax
from jax.experimental import pallas as pl
from jax.experimental.pallas import tpu as pltpu
from jax.experimental.pallas import tpu_sc as plsc

_NUM_WORKERS = 32


def _sc_gather_rows(table3, idx):
    _, V, D = table3.shape
    B = idx.shape[0]
    bpw = B // _NUM_WORKERS
    mesh = plsc.VectorSubcoreMesh(core_axis_name="c", subcore_axis_name="s")
    cp = pltpu.CompilerParams()
    if "needs_layout_passes" in pltpu.CompilerParams.__dataclass_fields__:
        cp = dataclasses.replace(cp, needs_layout_passes=False)

    @functools.partial(
        pl.kernel,
        mesh=mesh,
        compiler_params=cp,
        out_type=jax.ShapeDtypeStruct((B, D), jnp.float32),
        scratch_types=[
            pltpu.VMEM((bpw,), jnp.int32),
            pltpu.VMEM((bpw, D), jnp.float32),
            pltpu.SemaphoreType.DMA,
        ],
    )
    def k(t_hbm, i_hbm, out_hbm, idx_v, rows_v, sem):
        t2 = t_hbm.at[0]
        wid = lax.axis_index("s") * 2 + lax.axis_index("c")
        base = wid * bpw
        pltpu.sync_copy(i_hbm.at[pl.ds(base, bpw)], idx_v)
        lane = lax.iota(jnp.int32, 16)

        @pl.loop(0, bpw, step=16)
        def _(j):
            v = idx_v[pl.ds(j, 16)]
            for u in range(16):
                s = jnp.sum(jnp.where(lane == u, v, 0))
                pltpu.async_copy(t2.at[s], rows_v.at[j + u], sem)

        @pl.loop(0, bpw)
        def _(i):
            pltpu.make_async_copy(t2.at[0], rows_v.at[i], sem).wait()

        pltpu.sync_copy(rows_v, out_hbm.at[pl.ds(base, bpw)])

    return k(table3, idx)


def _mlp_body(m_ref, gidf_ref, gt_ref, w1_ref, b1_ref, w2_ref, b2_ref,
              w3_ref, b3_ref, o_ref):
    m_emb = m_ref[...]
    blk = m_emb.shape[0]
    P = gt_ref.shape[0]
    lane = lax.broadcasted_iota(jnp.int32, (blk, P), 1).astype(jnp.float32)
    onehot = jnp.where(lane == gidf_ref[...], 1.0, 0.0)
    g_emb = jnp.dot(onehot, gt_ref[...], preferred_element_type=jnp.float32)
    x = m_emb + g_emb
    h = jnp.dot(x, w1_ref[...], preferred_element_type=jnp.float32)
    h = jnp.maximum(h + b1_ref[...], 0.0)
    h = jnp.dot(h, w2_ref[...], preferred_element_type=jnp.float32)
    h = jnp.maximum(h + b2_ref[...], 0.0)
    logits = jnp.dot(h, w3_ref[...], preferred_element_type=jnp.float32)
    logits = logits + b3_ref[...]
    mx = jnp.max(logits, axis=-1, keepdims=True)
    e = jnp.exp(logits - mx)
    o_ref[...] = e / jnp.sum(e, axis=-1, keepdims=True)


def _mlp_softmax(m_emb, gid_f, gt_pad, W1, b1, W2, b2, W3p, b3p, blk=2048):
    B, D = m_emb.shape
    GP = gt_pad.shape[0]
    H1 = W1.shape[1]
    H2 = W2.shape[1]
    P = W3p.shape[1]
    return pl.pallas_call(
        _mlp_body,
        grid=(B // blk,),
        in_specs=[
            pl.BlockSpec((blk, D), lambda i: (i, 0)),
            pl.BlockSpec((blk, 1), lambda i: (i, 0)),
            pl.BlockSpec((GP, D), lambda i: (0, 0)),
            pl.BlockSpec((D, H1), lambda i: (0, 0)),
            pl.BlockSpec((1, H1), lambda i: (0, 0)),
            pl.BlockSpec((H1, H2), lambda i: (0, 0)),
            pl.BlockSpec((1, H2), lambda i: (0, 0)),
            pl.BlockSpec((H2, P), lambda i: (0, 0)),
            pl.BlockSpec((1, P), lambda i: (0, 0)),
        ],
        out_specs=pl.BlockSpec((blk, P), lambda i: (i, 0)),
        out_shape=jax.ShapeDtypeStruct((B, P), jnp.float32),
    )(m_emb, gid_f, gt_pad, W1, b1.reshape(1, H1), W2, b2.reshape(1, H2),
      W3p, b3p.reshape(1, P))


def kernel(movie_id, genre_id, movie_table, genre_table, W1, b1, W2, b2, W3,
           b3):
    B = movie_id.shape[0]
    V, D = movie_table.shape
    m_emb = _sc_gather_rows(movie_table.reshape(1, V, D),
                            movie_id.astype(jnp.int32))
    gid_f = genre_id.astype(jnp.float32).reshape(B, 1)
    G = genre_table.shape[0]
    gt_pad = jnp.pad(genre_table, ((0, (-G) % 128), (0, 0)))
    NG = W3.shape[1]
    pad = (-NG) % 128
    W3p = jnp.pad(W3, ((0, 0), (0, pad)))
    b3p = jnp.concatenate([b3, jnp.full((pad,), -1e30, dtype=b3.dtype)])
    out = _mlp_softmax(m_emb, gid_f, gt_pad, W1, b1, W2, b2, W3p, b3p)
    return out[:, :NG]
